# baseline (speedup 1.0000x reference)
import jax
import jax.numpy as jnp
from jax.experimental import pallas as pl

_K = 8192
_H = 32
_TILE = 256
_CHUNK = 4096


def _vq_body(z_ref, emb_ref, z2_ref, e2_ref, zq_ref, idx_ref, loss_ref):
    i = pl.program_id(0)
    z = z_ref[...]
    e = emb_ref[...]
    z2 = z2_ref[...]
    e2 = e2_ref[...]
    lhs = (2.0 * z).astype(jnp.bfloat16)
    rhs = e.astype(jnp.bfloat16)
    conv = jax.lax.dot_general(lhs, rhs, (((1,), (1,)), ((), ())),
                               preferred_element_type=jnp.float32)
    dist = (z2 - conv) + e2

    iota = jax.lax.broadcasted_iota(jnp.int32, (_TILE, _CHUNK), 1)
    run_v = None
    run_i = None
    for c0 in range(0, _K, _CHUNK):
        blk = dist[:, c0:c0 + _CHUNK]
        m = jnp.min(blk, axis=1, keepdims=True)
        a = jnp.min(jnp.where(blk == m, iota + c0, _K), axis=1,
                    keepdims=True)
        if run_v is None:
            run_v, run_i = m, a
        else:
            rb = run_v.astype(jnp.bfloat16).astype(jnp.float32)
            keep_old = (rb < m) | ((rb == m) & (run_i < a))
            run_v = jnp.where(keep_old, rb, m)
            run_i = jnp.where(keep_old, run_i, a)

    idx = run_i[:, 0]

    iota_full = jax.lax.broadcasted_iota(jnp.int32, (_TILE, _K), 1)
    onehot = (iota_full == run_i).astype(jnp.float32)
    zq = jax.lax.dot_general(onehot, e, (((1,), (0,)), ((), ())),
                             precision=jax.lax.Precision.HIGHEST,
                             preferred_element_type=jnp.float32)
    diff = zq - z
    zq_ref[...] = z + diff
    idx_ref[0, 0, :] = idx
    part = jnp.sum(diff * diff).reshape(1, 1)

    @pl.when(i == 0)
    def _init():
        loss_ref[...] = part

    @pl.when(i != 0)
    def _acc():
        loss_ref[...] = loss_ref[...] + part


def kernel(z, emb):
    b, t, h = z.shape
    n = b * t
    flat = z.reshape(n, h)
    ntiles = n // _TILE
    z2 = jnp.sum(flat ** 2, axis=1, keepdims=True)
    e2t = jnp.sum(emb ** 2, axis=1, keepdims=True).T

    zq, idx3, loss_sum = pl.pallas_call(
        _vq_body,
        grid=(ntiles,),
        in_specs=[
            pl.BlockSpec((_TILE, h), lambda i: (i, 0)),
            pl.BlockSpec((_K, h), lambda i: (0, 0)),
            pl.BlockSpec((_TILE, 1), lambda i: (i, 0)),
            pl.BlockSpec((1, _K), lambda i: (0, 0)),
        ],
        out_specs=[
            pl.BlockSpec((_TILE, h), lambda i: (i, 0)),
            pl.BlockSpec((1, 1, _TILE), lambda i: (i, 0, 0)),
            pl.BlockSpec((1, 1), lambda i: (0, 0)),
        ],
        out_shape=[
            jax.ShapeDtypeStruct((n, h), jnp.float32),
            jax.ShapeDtypeStruct((ntiles, 1, _TILE), jnp.int32),
            jax.ShapeDtypeStruct((1, 1), jnp.float32),
        ],
    )(flat, emb, z2, e2t)

    z_st = zq.reshape(b, t, h)
    idx = idx3.reshape(b, t)
    loss = loss_sum[0, 0] / jnp.float32(n * h)
    return (z_st, idx, loss, loss)


# speedup vs baseline: 3.1877x; 3.1877x over previous
import jax
import jax.numpy as jnp
from jax.experimental import pallas as pl
from jax.experimental.pallas import tpu as pltpu
from jax.experimental.pallas import tpu_sc as plsc

_K = 8192
_H = 32
_TILE = 256
_CHUNK = 4096
_GW = 128


def _vq_body(lhs_ref, rhs_ref, z2_ref, e2_ref, idx_ref, loss_ref):
    i = pl.program_id(0)
    lhs = lhs_ref[...]
    rhs = rhs_ref[...]
    z2 = z2_ref[...]
    e2 = e2_ref[...]
    conv = jax.lax.dot_general(lhs, rhs, (((1,), (1,)), ((), ())),
                               preferred_element_type=jnp.float32)
    dist = (z2 - conv) + e2

    iota = jax.lax.broadcasted_iota(jnp.int32, (_TILE, _CHUNK), 1)
    run_v = None
    run_i = None
    chosen_v = None
    for c0 in range(0, _K, _CHUNK):
        blk = dist[:, c0:c0 + _CHUNK]
        m = jnp.min(blk, axis=1, keepdims=True)
        a = jnp.min(jnp.where(blk == m, iota + c0, _K), axis=1,
                    keepdims=True)
        if run_v is None:
            run_v, run_i, chosen_v = m, a, m
        else:
            rb = run_v.astype(jnp.bfloat16).astype(jnp.float32)
            keep_old = (rb < m) | ((rb == m) & (run_i < a))
            run_v = jnp.where(keep_old, rb, m)
            run_i = jnp.where(keep_old, run_i, a)
            chosen_v = jnp.where(keep_old, chosen_v, m)

    idx_ref[0, 0, :] = run_i[:, 0]
    part = jnp.sum(chosen_v).reshape(1, 1)

    @pl.when(i == 0)
    def _init():
        loss_ref[...] = part

    @pl.when(i != 0)
    def _acc():
        loss_ref[...] = loss_ref[...] + part


def _sc_gather(emb_pad, idx_row):
    n = idx_row.shape[1]
    w = emb_pad.shape[1]
    mesh = plsc.VectorSubcoreMesh(core_axis_name="core",
                                  subcore_axis_name="subcore")

    @pl.kernel(out_type=jax.ShapeDtypeStruct((n, w), jnp.float32), mesh=mesh)
    def k(emb_hbm, i_hbm, o_hbm):
        def body(i_vmem, o_vmem):
            pltpu.sync_copy(emb_hbm.at[i_vmem.at[0]], o_vmem)

        pltpu.emit_pipeline(
            body,
            grid=(n // _GW,),
            in_specs=[pl.BlockSpec((1, _GW), index_map=lambda i: (0, i))],
            out_specs=[pl.BlockSpec((_GW, w), index_map=lambda i: (i, 0))],
            core_axis_name=("core", "subcore"),
            dimension_semantics=(pltpu.PARALLEL,),
        )(i_hbm, o_hbm)

    return k(emb_pad, idx_row)


def kernel(z, emb):
    b, t, h = z.shape
    n = b * t
    flat = z.reshape(n, h)
    ntiles = n // _TILE
    z2 = jnp.sum(flat ** 2, axis=1, keepdims=True)
    e2t = jnp.sum(emb ** 2, axis=1, keepdims=True).T
    lhs = (2.0 * flat).astype(jnp.bfloat16)
    rhs = emb.astype(jnp.bfloat16)

    idx3, loss_sum = pl.pallas_call(
        _vq_body,
        grid=(ntiles,),
        in_specs=[
            pl.BlockSpec((_TILE, h), lambda i: (i, 0)),
            pl.BlockSpec((_K, h), lambda i: (0, 0)),
            pl.BlockSpec((_TILE, 1), lambda i: (i, 0)),
            pl.BlockSpec((1, _K), lambda i: (0, 0)),
        ],
        out_specs=[
            pl.BlockSpec((1, 1, _TILE), lambda i: (i, 0, 0)),
            pl.BlockSpec((1, 1), lambda i: (0, 0)),
        ],
        out_shape=[
            jax.ShapeDtypeStruct((ntiles, 1, _TILE), jnp.int32),
            jax.ShapeDtypeStruct((1, 1), jnp.float32),
        ],
    )(lhs, rhs, z2, e2t)

    emb_pad = jnp.pad(emb, ((0, 0), (0, 128 - h)))
    zq = _sc_gather(emb_pad, idx3.reshape(1, n))[:, :h]

    z_st = zq.reshape(b, t, h)
    idx = idx3.reshape(b, t)
    loss = loss_sum[0, 0] / jnp.float32(n * h)
    return (z_st, idx, loss, loss)


# speedup vs baseline: 3.6623x; 1.1489x over previous
import jax
import jax.numpy as jnp
from jax.experimental import pallas as pl
from jax.experimental.pallas import tpu as pltpu
from jax.experimental.pallas import tpu_sc as plsc

_K = 8192
_H = 32
_TILE = 256
_CHUNK = 4096
_GW = 128


def _vq_body(lhs_ref, rhs_ref, z2_ref, e2_ref, idx_ref, loss_ref):
    i = pl.program_id(0)
    lhs = lhs_ref[...]
    rhs = rhs_ref[...]
    z2 = z2_ref[...]
    e2 = e2_ref[...]
    conv = jax.lax.dot_general(lhs, rhs, (((1,), (1,)), ((), ())),
                               preferred_element_type=jnp.float32)

    lane = jax.lax.broadcasted_iota(jnp.int32, (_TILE, 128), 1)
    run_v = None
    run_i = None
    chosen_v = None
    for c0 in range(0, _K, _CHUNK):
        acc_v = None
        acc_g = None
        for g in range(_CHUNK // 128):
            s = c0 + g * 128
            blk_g = (z2 - conv[:, s:s + 128]) + e2[:, s:s + 128]
            if acc_v is None:
                acc_v = blk_g
                acc_g = jnp.zeros((_TILE, 128), jnp.int32)
            else:
                lt = blk_g < acc_v
                acc_v = jnp.where(lt, blk_g, acc_v)
                acc_g = jnp.where(lt, g, acc_g)
        m = jnp.min(acc_v, axis=1, keepdims=True)
        k = acc_g * 128 + lane
        a = jnp.min(jnp.where(acc_v == m, k, _CHUNK), axis=1,
                    keepdims=True) + c0
        if run_v is None:
            run_v, run_i, chosen_v = m, a, m
        else:
            rb = run_v.astype(jnp.bfloat16).astype(jnp.float32)
            keep_old = (rb < m) | ((rb == m) & (run_i < a))
            run_v = jnp.where(keep_old, rb, m)
            run_i = jnp.where(keep_old, run_i, a)
            chosen_v = jnp.where(keep_old, chosen_v, m)

    idx_ref[0, 0, :] = run_i[:, 0]
    part = jnp.sum(chosen_v).reshape(1, 1)

    @pl.when(i == 0)
    def _init():
        loss_ref[...] = part

    @pl.when(i != 0)
    def _acc():
        loss_ref[...] = loss_ref[...] + part


def _sc_gather(emb_pad, idx_row):
    n = idx_row.shape[1]
    w = emb_pad.shape[1]
    mesh = plsc.VectorSubcoreMesh(core_axis_name="core",
                                  subcore_axis_name="subcore")

    @pl.kernel(out_type=jax.ShapeDtypeStruct((n, w), jnp.float32), mesh=mesh)
    def k(emb_hbm, i_hbm, o_hbm):
        def body(i_vmem, o_vmem):
            pltpu.sync_copy(emb_hbm.at[i_vmem.at[0]], o_vmem)

        pltpu.emit_pipeline(
            body,
            grid=(n // _GW,),
            in_specs=[pl.BlockSpec((1, _GW), index_map=lambda i: (0, i))],
            out_specs=[pl.BlockSpec((_GW, w), index_map=lambda i: (i, 0))],
            core_axis_name=("core", "subcore"),
            dimension_semantics=(pltpu.PARALLEL,),
        )(i_hbm, o_hbm)

    return k(emb_pad, idx_row)


def kernel(z, emb):
    b, t, h = z.shape
    n = b * t
    flat = z.reshape(n, h)
    ntiles = n // _TILE
    z2 = jnp.sum(flat ** 2, axis=1, keepdims=True)
    e2t = jnp.sum(emb ** 2, axis=1, keepdims=True).T
    lhs = (2.0 * flat).astype(jnp.bfloat16)
    rhs = emb.astype(jnp.bfloat16)

    idx3, loss_sum = pl.pallas_call(
        _vq_body,
        grid=(ntiles,),
        in_specs=[
            pl.BlockSpec((_TILE, h), lambda i: (i, 0)),
            pl.BlockSpec((_K, h), lambda i: (0, 0)),
            pl.BlockSpec((_TILE, 1), lambda i: (i, 0)),
            pl.BlockSpec((1, _K), lambda i: (0, 0)),
        ],
        out_specs=[
            pl.BlockSpec((1, 1, _TILE), lambda i: (i, 0, 0)),
            pl.BlockSpec((1, 1), lambda i: (0, 0)),
        ],
        out_shape=[
            jax.ShapeDtypeStruct((ntiles, 1, _TILE), jnp.int32),
            jax.ShapeDtypeStruct((1, 1), jnp.float32),
        ],
    )(lhs, rhs, z2, e2t)

    emb_pad = jnp.pad(emb, ((0, 0), (0, 128 - h)))
    zq = _sc_gather(emb_pad, idx3.reshape(1, n))[:, :h]

    z_st = zq.reshape(b, t, h)
    idx = idx3.reshape(b, t)
    loss = loss_sum[0, 0] / jnp.float32(n * h)
    return (z_st, idx, loss, loss)


# speedup vs baseline: 3.9494x; 1.0784x over previous
import jax
import jax.numpy as jnp
from jax.experimental import pallas as pl
from jax.experimental.pallas import tpu as pltpu
from jax.experimental.pallas import tpu_sc as plsc

_K = 8192
_H = 32
_TILE = 512
_CHUNK = 4096
_GW = 128


def _vq_body(lhs_ref, rhs_ref, z2_ref, e2_ref, idx_ref, loss_ref):
    i = pl.program_id(0)
    lhs = lhs_ref[...]
    rhs = rhs_ref[...]
    z2 = z2_ref[...]
    e2 = e2_ref[...]
    conv = jax.lax.dot_general(lhs, rhs, (((1,), (1,)), ((), ())),
                               preferred_element_type=jnp.float32)

    lane = jax.lax.broadcasted_iota(jnp.int32, (_TILE, 128), 1)
    run_v = None
    run_i = None
    chosen_v = None
    for c0 in range(0, _K, _CHUNK):
        acc_v = None
        acc_g = None
        for g in range(_CHUNK // 128):
            s = c0 + g * 128
            blk_g = (z2 - conv[:, s:s + 128]) + e2[:, s:s + 128]
            if acc_v is None:
                acc_v = blk_g
                acc_g = jnp.zeros((_TILE, 128), jnp.int32)
            else:
                lt = blk_g < acc_v
                acc_v = jnp.where(lt, blk_g, acc_v)
                acc_g = jnp.where(lt, g, acc_g)
        m = jnp.min(acc_v, axis=1, keepdims=True)
        k = acc_g * 128 + lane
        a = jnp.min(jnp.where(acc_v == m, k, _CHUNK), axis=1,
                    keepdims=True) + c0
        if run_v is None:
            run_v, run_i, chosen_v = m, a, m
        else:
            rb = run_v.astype(jnp.bfloat16).astype(jnp.float32)
            keep_old = (rb < m) | ((rb == m) & (run_i < a))
            run_v = jnp.where(keep_old, rb, m)
            run_i = jnp.where(keep_old, run_i, a)
            chosen_v = jnp.where(keep_old, chosen_v, m)

    idx_ref[0, 0, :] = run_i[:, 0]
    part = jnp.sum(chosen_v).reshape(1, 1)

    @pl.when(i == 0)
    def _init():
        loss_ref[...] = part

    @pl.when(i != 0)
    def _acc():
        loss_ref[...] = loss_ref[...] + part


def _sc_gather(emb_pad, idx_row):
    n = idx_row.shape[1]
    w = emb_pad.shape[1]
    mesh = plsc.VectorSubcoreMesh(core_axis_name="core",
                                  subcore_axis_name="subcore")

    @pl.kernel(out_type=jax.ShapeDtypeStruct((n, w), jnp.float32), mesh=mesh)
    def k(emb_hbm, i_hbm, o_hbm):
        def body(i_vmem, o_vmem):
            pltpu.sync_copy(emb_hbm.at[i_vmem.at[0]], o_vmem)

        pltpu.emit_pipeline(
            body,
            grid=(n // _GW,),
            in_specs=[pl.BlockSpec((1, _GW), index_map=lambda i: (0, i))],
            out_specs=[pl.BlockSpec((_GW, w), index_map=lambda i: (i, 0))],
            core_axis_name=("core", "subcore"),
            dimension_semantics=(pltpu.PARALLEL,),
        )(i_hbm, o_hbm)

    return k(emb_pad, idx_row)


def kernel(z, emb):
    b, t, h = z.shape
    n = b * t
    flat = z.reshape(n, h)
    ntiles = n // _TILE
    z2 = jnp.sum(flat ** 2, axis=1, keepdims=True)
    e2t = jnp.sum(emb ** 2, axis=1, keepdims=True).T
    lhs = (2.0 * flat).astype(jnp.bfloat16)
    rhs = emb.astype(jnp.bfloat16)

    idx3, loss_sum = pl.pallas_call(
        _vq_body,
        grid=(ntiles,),
        in_specs=[
            pl.BlockSpec((_TILE, h), lambda i: (i, 0)),
            pl.BlockSpec((_K, h), lambda i: (0, 0)),
            pl.BlockSpec((_TILE, 1), lambda i: (i, 0)),
            pl.BlockSpec((1, _K), lambda i: (0, 0)),
        ],
        out_specs=[
            pl.BlockSpec((1, 1, _TILE), lambda i: (i, 0, 0)),
            pl.BlockSpec((1, 1), lambda i: (0, 0)),
        ],
        out_shape=[
            jax.ShapeDtypeStruct((ntiles, 1, _TILE), jnp.int32),
            jax.ShapeDtypeStruct((1, 1), jnp.float32),
        ],
    )(lhs, rhs, z2, e2t)

    emb_pad = jnp.pad(emb, ((0, 0), (0, 128 - h)))
    zq = _sc_gather(emb_pad, idx3.reshape(1, n))[:, :h]

    z_st = zq.reshape(b, t, h)
    idx = idx3.reshape(b, t)
    loss = loss_sum[0, 0] / jnp.float32(n * h)
    return (z_st, idx, loss, loss)
